# initial kernel scaffold (unmeasured)
import jax
import jax.numpy as jnp
from jax import lax
from jax.experimental import pallas as pl
from jax.experimental.pallas import tpu as pltpu

N_DEV = 16
SQ = 1024
SKV = 1024
HQ_LOC = 8
DH = 128
D_LOC = HQ_LOC * DH
CHUNK = SQ // N_DEV
N_HOPS = N_DEV - 1
SCALE = 0.08838834764831843


def kernel(x, Wq, K_ext, V_ext, Wo):
    def body(
        x_ref, wq_ref, k_hbm, v_hbm, wo_ref, out_ref,
        q_buf, k_buf, v_buf, ctx_buf, partial_buf, send_buf, recv_buf,
        local_sems, rs_send_sems, rs_recv_sems, ag_send_sems, ag_recv_sems,
    ):
        p = lax.axis_index("i")
        left = lax.rem(p + N_DEV - 1, N_DEV)
        right = lax.rem(p + 1, N_DEV)

        barrier_sem = pltpu.get_barrier_semaphore()
        for nbr in (left, right):
            pl.semaphore_signal(
                barrier_sem, inc=1,
                device_id=(nbr,), device_id_type=pl.DeviceIdType.MESH,
            )
        pl.semaphore_wait(barrier_sem, 2)

        head0 = p * HQ_LOC
        copies = []
        for h in range(HQ_LOC):
            ck = pltpu.make_async_copy(
                k_hbm.at[0, :, head0 + h, :], k_buf.at[h], local_sems.at[h])
            cv = pltpu.make_async_copy(
                v_hbm.at[0, :, head0 + h, :], v_buf.at[h],
                local_sems.at[HQ_LOC + h])
            ck.start()
            cv.start()
            copies.append((ck, cv))

        q_buf[...] = jnp.dot(
            x_ref[0], wq_ref[...], preferred_element_type=jnp.float32)

        for ck, cv in copies:
            ck.wait()
            cv.wait()

        qi = lax.broadcasted_iota(jnp.int32, (SQ, SKV), 0)
        ki = lax.broadcasted_iota(jnp.int32, (SQ, SKV), 1)
        mask = (jnp.abs(qi - ki) <= 128) | (ki < 32) | (qi < 32)

        for h in range(HQ_LOC):
            qh = q_buf[:, h * DH:(h + 1) * DH]
            scores = lax.dot_general(
                qh, k_buf[h],
                dimension_numbers=(((1,), (1,)), ((), ())),
                preferred_element_type=jnp.float32,
            ) * SCALE
            scores = jnp.where(mask, scores, -1e9)
            smax = jnp.max(scores, axis=-1, keepdims=True)
            w = jnp.exp(scores - smax)
            w = w / jnp.sum(w, axis=-1, keepdims=True)
            ctx_buf[:, h * DH:(h + 1) * DH] = jnp.dot(
                w, v_buf[h], preferred_element_type=jnp.float32)

        partial_buf[...] = jnp.dot(
            ctx_buf[...], wo_ref[...], preferred_element_type=jnp.float32)

        for s in range(N_HOPS):
            cidx = lax.rem(p + 2 * N_DEV - s, N_DEV)
            rows = pl.ds(cidx * CHUNK, CHUNK)
            if s == 0:
                src = partial_buf.at[rows, :]
            else:
                send_buf[s, :, :] = recv_buf[s - 1] + partial_buf[rows, :]
                src = send_buf.at[s]
            rdma = pltpu.make_async_remote_copy(
                src_ref=src,
                dst_ref=recv_buf.at[s],
                send_sem=rs_send_sems.at[s],
                recv_sem=rs_recv_sems.at[s],
                device_id=(right,),
                device_id_type=pl.DeviceIdType.MESH,
            )
            rdma.start()
            rdma.wait()

        own = lax.rem(p + 1, N_DEV)
        own_rows = pl.ds(own * CHUNK, CHUNK)
        out_ref[0, own_rows, :] = (
            recv_buf[N_HOPS - 1] + partial_buf[own_rows, :])

        for t in range(N_HOPS):
            sc = lax.rem(p + 1 + 2 * N_DEV - t, N_DEV)
            srows = pl.ds(sc * CHUNK, CHUNK)
            rdma = pltpu.make_async_remote_copy(
                src_ref=out_ref.at[0, srows, :],
                dst_ref=out_ref.at[0, srows, :],
                send_sem=ag_send_sems.at[t],
                recv_sem=ag_recv_sems.at[t],
                device_id=(right,),
                device_id_type=pl.DeviceIdType.MESH,
            )
            rdma.start()
            rdma.wait()

    return pl.pallas_call(
        body,
        out_shape=jax.ShapeDtypeStruct((1, SQ, SQ), jnp.float32),
        in_specs=[
            pl.BlockSpec(memory_space=pltpu.VMEM),
            pl.BlockSpec(memory_space=pltpu.VMEM),
            pl.BlockSpec(memory_space=pltpu.ANY),
            pl.BlockSpec(memory_space=pltpu.ANY),
            pl.BlockSpec(memory_space=pltpu.VMEM),
        ],
        out_specs=pl.BlockSpec(memory_space=pltpu.VMEM),
        scratch_shapes=[
            pltpu.VMEM((SQ, D_LOC), jnp.float32),
            pltpu.VMEM((HQ_LOC, SKV, DH), jnp.float32),
            pltpu.VMEM((HQ_LOC, SKV, DH), jnp.float32),
            pltpu.VMEM((SQ, D_LOC), jnp.float32),
            pltpu.VMEM((SQ, SQ), jnp.float32),
            pltpu.VMEM((N_HOPS, CHUNK, SQ), jnp.float32),
            pltpu.VMEM((N_HOPS, CHUNK, SQ), jnp.float32),
            pltpu.SemaphoreType.DMA((2 * HQ_LOC,)),
            pltpu.SemaphoreType.DMA((N_HOPS,)),
            pltpu.SemaphoreType.DMA((N_HOPS,)),
            pltpu.SemaphoreType.DMA((N_HOPS,)),
            pltpu.SemaphoreType.DMA((N_HOPS,)),
        ],
        compiler_params=pltpu.CompilerParams(collective_id=0),
    )(x, Wq, K_ext, V_ext, Wo)


# baseline (device time: 174126 ns/iter reference)
import jax
import jax.numpy as jnp
from jax import lax
from jax.experimental import pallas as pl
from jax.experimental.pallas import tpu as pltpu

N_DEV = 16
SQ = 1024
SKV = 1024
HQ_LOC = 8
DH = 128
D_LOC = HQ_LOC * DH
CHUNK = SQ // N_DEV
N_HOPS = N_DEV - 1
SCALE = 0.08838834764831843


def kernel(x, Wq, K_ext, V_ext, Wo):
    def body(
        x_ref, wq_ref, k_hbm, v_hbm, wo_ref, out_ref,
        q_buf, k_buf, v_buf, ctx_buf, partial_buf, send_buf, recv_buf,
        local_sems, rs_send_sems, rs_recv_sems, ag_send_sems, ag_recv_sems,
    ):
        p = lax.axis_index("i")
        left = lax.rem(p + N_DEV - 1, N_DEV)
        right = lax.rem(p + 1, N_DEV)

        barrier_sem = pltpu.get_barrier_semaphore()
        for nbr in (left, right):
            pl.semaphore_signal(
                barrier_sem, inc=1,
                device_id=(nbr,), device_id_type=pl.DeviceIdType.MESH,
            )
        pl.semaphore_wait(barrier_sem, 2)

        head0 = p * HQ_LOC
        copies = []
        for h in range(HQ_LOC):
            ck = pltpu.make_async_copy(
                k_hbm.at[0, :, head0 + h, :], k_buf.at[h], local_sems.at[h])
            cv = pltpu.make_async_copy(
                v_hbm.at[0, :, head0 + h, :], v_buf.at[h],
                local_sems.at[HQ_LOC + h])
            ck.start()
            cv.start()
            copies.append((ck, cv))

        q_buf[...] = jnp.dot(
            x_ref[0], wq_ref[...], preferred_element_type=jnp.float32)

        for ck, cv in copies:
            ck.wait()
            cv.wait()

        qi = lax.broadcasted_iota(jnp.int32, (SQ, SKV), 0)
        ki = lax.broadcasted_iota(jnp.int32, (SQ, SKV), 1)
        mask = (jnp.abs(qi - ki) <= 128) | (ki < 32) | (qi < 32)

        for h in range(HQ_LOC):
            qh = q_buf[:, h * DH:(h + 1) * DH]
            scores = lax.dot_general(
                qh, k_buf[h],
                dimension_numbers=(((1,), (1,)), ((), ())),
                preferred_element_type=jnp.float32,
            ) * SCALE
            scores = jnp.where(mask, scores, -1e9)
            smax = jnp.max(scores, axis=-1, keepdims=True)
            w = jnp.exp(scores - smax)
            w = w / jnp.sum(w, axis=-1, keepdims=True)
            ctx_buf[:, h * DH:(h + 1) * DH] = jnp.dot(
                w, v_buf[h], preferred_element_type=jnp.float32)

        partial_buf[...] = jnp.dot(
            ctx_buf[...], wo_ref[...], preferred_element_type=jnp.float32)

        for s in range(N_HOPS):
            cidx = lax.rem(p + 2 * N_DEV - s, N_DEV)
            rows = pl.ds(cidx * CHUNK, CHUNK)
            if s == 0:
                src = partial_buf.at[rows, :]
            else:
                send_buf[s, :, :] = recv_buf[s - 1] + partial_buf[rows, :]
                src = send_buf.at[s]
            rdma = pltpu.make_async_remote_copy(
                src_ref=src,
                dst_ref=recv_buf.at[s],
                send_sem=rs_send_sems.at[s],
                recv_sem=rs_recv_sems.at[s],
                device_id=(right,),
                device_id_type=pl.DeviceIdType.MESH,
            )
            rdma.start()
            rdma.wait()

        own = lax.rem(p + 1, N_DEV)
        own_rows = pl.ds(own * CHUNK, CHUNK)
        out_ref[0, own_rows, :] = (
            recv_buf[N_HOPS - 1] + partial_buf[own_rows, :])

        for t in range(N_HOPS):
            sc = lax.rem(p + 1 + 2 * N_DEV - t, N_DEV)
            srows = pl.ds(sc * CHUNK, CHUNK)
            rdma = pltpu.make_async_remote_copy(
                src_ref=out_ref.at[0, srows, :],
                dst_ref=out_ref.at[0, srows, :],
                send_sem=ag_send_sems.at[t],
                recv_sem=ag_recv_sems.at[t],
                device_id=(right,),
                device_id_type=pl.DeviceIdType.MESH,
            )
            rdma.start()
            rdma.wait()

    return pl.pallas_call(
        body,
        out_shape=jax.ShapeDtypeStruct((1, SQ, SQ), jnp.float32),
        in_specs=[
            pl.BlockSpec(memory_space=pltpu.VMEM),
            pl.BlockSpec(memory_space=pltpu.VMEM),
            pl.BlockSpec(memory_space=pl.ANY),
            pl.BlockSpec(memory_space=pl.ANY),
            pl.BlockSpec(memory_space=pltpu.VMEM),
        ],
        out_specs=pl.BlockSpec(memory_space=pltpu.VMEM),
        scratch_shapes=[
            pltpu.VMEM((SQ, D_LOC), jnp.float32),
            pltpu.VMEM((HQ_LOC, SKV, DH), jnp.float32),
            pltpu.VMEM((HQ_LOC, SKV, DH), jnp.float32),
            pltpu.VMEM((SQ, D_LOC), jnp.float32),
            pltpu.VMEM((SQ, SQ), jnp.float32),
            pltpu.VMEM((N_HOPS, CHUNK, SQ), jnp.float32),
            pltpu.VMEM((N_HOPS, CHUNK, SQ), jnp.float32),
            pltpu.SemaphoreType.DMA((2 * HQ_LOC,)),
            pltpu.SemaphoreType.DMA((N_HOPS,)),
            pltpu.SemaphoreType.DMA((N_HOPS,)),
            pltpu.SemaphoreType.DMA((N_HOPS,)),
            pltpu.SemaphoreType.DMA((N_HOPS,)),
        ],
        compiler_params=pltpu.CompilerParams(collective_id=0),
    )(x, Wq, K_ext, V_ext, Wo)


# device time: 129330 ns/iter; 1.3464x vs baseline; 1.3464x over previous
import jax
import jax.numpy as jnp
from jax import lax
from jax.experimental import pallas as pl
from jax.experimental.pallas import tpu as pltpu

N_DEV = 16
SQ = 1024
SKV = 1024
HQ_LOC = 8
DH = 128
D_LOC = HQ_LOC * DH
PC = SQ // 4
SC = PC // 4
SCALE = 0.08838834764831843
NEG = -1e9


def kernel(x, Wq, K_ext, V_ext, Wo):
    def body(
        x_ref, wq_ref, k_hbm, v_hbm, wo_ref, out_ref,
        q_chunk, ctx_chunk, k_buf, v_buf, partial_buf,
        send_a, recv_a, strip_buf, send_b, recv_b,
        local_sems, a_send, a_recv, b_send, b_recv,
        c_send, c_recv, d_send, d_recv,
    ):
        p = lax.axis_index("i")
        q = lax.rem(p, 4)
        r = p // 4
        next_q = r * 4 + lax.rem(q + 1, 4)
        prev_q = r * 4 + lax.rem(q + 3, 4)
        next_r = lax.rem(r + 1, 4) * 4 + q
        prev_r = lax.rem(r + 3, 4) * 4 + q

        barrier_sem = pltpu.get_barrier_semaphore()
        for nbr in (next_q, prev_q, next_r, prev_r):
            pl.semaphore_signal(
                barrier_sem, inc=1,
                device_id=(nbr,), device_id_type=pl.DeviceIdType.MESH,
            )
        pl.semaphore_wait(barrier_sem, 4)

        head0 = p * HQ_LOC
        copies = []
        for h in range(HQ_LOC):
            ck = pltpu.make_async_copy(
                k_hbm.at[0, :, head0 + h, :], k_buf.at[h], local_sems.at[h])
            cv = pltpu.make_async_copy(
                v_hbm.at[0, :, head0 + h, :], v_buf.at[h],
                local_sems.at[HQ_LOC + h])
            ck.start()
            cv.start()
            copies.append((ck, cv))
        for ck, cv in copies:
            ck.wait()
            cv.wait()

        ki = lax.broadcasted_iota(jnp.int32, (PC, SKV), 1)

        def compute_chunk(c):
            rows = pl.ds(c * PC, PC)
            q_chunk[...] = jnp.dot(
                x_ref[0, rows, :], wq_ref[...],
                preferred_element_type=jnp.float32)
            qi = lax.broadcasted_iota(jnp.int32, (PC, SKV), 0) + c * PC
            mask = (jnp.abs(qi - ki) <= 128) | (ki < 32) | (qi < 32)
            for h in range(HQ_LOC):
                scores = lax.dot_general(
                    q_chunk[:, h * DH:(h + 1) * DH], k_buf[h],
                    dimension_numbers=(((1,), (1,)), ((), ())),
                    preferred_element_type=jnp.float32,
                ) * SCALE
                scores = jnp.where(mask, scores, NEG)
                smax = jnp.max(scores, axis=-1, keepdims=True)
                w = jnp.exp(scores - smax)
                w = w / jnp.sum(w, axis=-1, keepdims=True)
                ctx_chunk[:, h * DH:(h + 1) * DH] = jnp.dot(
                    w, v_buf[h], preferred_element_type=jnp.float32)
            partial_buf[rows, :] = jnp.dot(
                ctx_chunk[...], wo_ref[...],
                preferred_element_type=jnp.float32)

        def ring_rdma(src, dst, send_sem, recv_sem, dev):
            return pltpu.make_async_remote_copy(
                src_ref=src, dst_ref=dst, send_sem=send_sem,
                recv_sem=recv_sem, device_id=(dev,),
                device_id_type=pl.DeviceIdType.MESH,
            )

        compute_chunk(q)
        rdmas_a = []
        rd = ring_rdma(
            partial_buf.at[pl.ds(q * PC, PC), :], recv_a.at[0],
            a_send.at[0], a_recv.at[0], next_q)
        rd.start()
        rdmas_a.append(rd)
        for s in (1, 2):
            cidx = lax.rem(q + 4 - s, 4)
            compute_chunk(cidx)
            rdmas_a[s - 1].wait()
            send_a[s, :, :] = recv_a[s - 1] + partial_buf[pl.ds(cidx * PC, PC), :]
            rd = ring_rdma(
                send_a.at[s], recv_a.at[s], a_send.at[s], a_recv.at[s], next_q)
            rd.start()
            rdmas_a.append(rd)
        oq = lax.rem(q + 1, 4)
        compute_chunk(oq)
        rdmas_a[2].wait()
        strip_buf[...] = recv_a[2] + partial_buf[pl.ds(oq * PC, PC), :]

        rdmas_b = []
        for t in range(3):
            sidx = lax.rem(r + 4 - t, 4)
            if t == 0:
                src = strip_buf.at[pl.ds(sidx * SC, SC), :]
            else:
                rdmas_b[t - 1].wait()
                send_b[t, :, :] = (
                    recv_b[t - 1] + strip_buf[pl.ds(sidx * SC, SC), :])
                src = send_b.at[t]
            rd = ring_rdma(src, recv_b.at[t], b_send.at[t], b_recv.at[t], next_r)
            rd.start()
            rdmas_b.append(rd)
        rdmas_b[2].wait()
        orr = lax.rem(r + 1, 4)
        strip_buf[pl.ds(orr * SC, SC), :] = (
            recv_b[2] + strip_buf[pl.ds(orr * SC, SC), :])

        for t in range(3):
            sidx = lax.rem(r + 5 - t, 4)
            srows = pl.ds(sidx * SC, SC)
            rd = ring_rdma(
                strip_buf.at[srows, :], strip_buf.at[srows, :],
                c_send.at[t], c_recv.at[t], next_r)
            rd.start()
            rd.wait()

        out_ref[0, pl.ds(oq * PC, PC), :] = strip_buf[...]

        for u in range(3):
            sidx = lax.rem(q + 5 - u, 4)
            srows = pl.ds(sidx * PC, PC)
            rd = ring_rdma(
                out_ref.at[0, srows, :], out_ref.at[0, srows, :],
                d_send.at[u], d_recv.at[u], next_q)
            rd.start()
            rd.wait()

    return pl.pallas_call(
        body,
        out_shape=jax.ShapeDtypeStruct((1, SQ, SQ), jnp.float32),
        in_specs=[
            pl.BlockSpec(memory_space=pltpu.VMEM),
            pl.BlockSpec(memory_space=pltpu.VMEM),
            pl.BlockSpec(memory_space=pl.ANY),
            pl.BlockSpec(memory_space=pl.ANY),
            pl.BlockSpec(memory_space=pltpu.VMEM),
        ],
        out_specs=pl.BlockSpec(memory_space=pltpu.VMEM),
        scratch_shapes=[
            pltpu.VMEM((PC, D_LOC), jnp.float32),
            pltpu.VMEM((PC, D_LOC), jnp.float32),
            pltpu.VMEM((HQ_LOC, SKV, DH), jnp.float32),
            pltpu.VMEM((HQ_LOC, SKV, DH), jnp.float32),
            pltpu.VMEM((SQ, SQ), jnp.float32),
            pltpu.VMEM((3, PC, SQ), jnp.float32),
            pltpu.VMEM((3, PC, SQ), jnp.float32),
            pltpu.VMEM((PC, SQ), jnp.float32),
            pltpu.VMEM((3, SC, SQ), jnp.float32),
            pltpu.VMEM((3, SC, SQ), jnp.float32),
            pltpu.SemaphoreType.DMA((2 * HQ_LOC,)),
            pltpu.SemaphoreType.DMA((3,)),
            pltpu.SemaphoreType.DMA((3,)),
            pltpu.SemaphoreType.DMA((3,)),
            pltpu.SemaphoreType.DMA((3,)),
            pltpu.SemaphoreType.DMA((3,)),
            pltpu.SemaphoreType.DMA((3,)),
            pltpu.SemaphoreType.DMA((3,)),
            pltpu.SemaphoreType.DMA((3,)),
        ],
        compiler_params=pltpu.CompilerParams(collective_id=0),
    )(x, Wq, K_ext, V_ext, Wo)


# device time: 104230 ns/iter; 1.6706x vs baseline; 1.2408x over previous
import jax
import jax.numpy as jnp
from jax import lax
from jax.experimental import pallas as pl
from jax.experimental.pallas import tpu as pltpu

N_DEV = 16
SQ = 1024
SKV = 1024
HQ_LOC = 8
DH = 128
D_LOC = HQ_LOC * DH
PC = SQ // 4
SC = PC // 4
CH = SQ // 2
WIN = 512
GB = 128
SCALE = 0.08838834764831843
NEG = -1e9

CW, CCW = 0, 1


def kernel(x, Wq, K_ext, V_ext, Wo):
    def body(
        x_ref, wq_ref, k_hbm, v_hbm, wo_ref, out_ref,
        q_chunk, ctx_chunk, k_buf, v_buf, partial_buf, pfix,
        send_a_cw, recv_a_cw, send_a_ccw, recv_a_ccw,
        strip_cw, strip_ccw, send_b_cw, recv_b_cw, send_b_ccw, recv_b_ccw,
        local_sems, a_send, a_recv, b_send, b_recv, c_send, c_recv,
        d_send, d_recv,
    ):
        p = lax.axis_index("i")
        q = lax.rem(p, 4)
        r = p // 4
        next_q = r * 4 + lax.rem(q + 1, 4)
        prev_q = r * 4 + lax.rem(q + 3, 4)
        next_r = lax.rem(r + 1, 4) * 4 + q
        prev_r = lax.rem(r + 3, 4) * 4 + q

        def rows4(idx):
            return pl.ds(lax.rem(idx, 4) * PC, PC)

        def subrows(idx):
            return pl.ds(lax.rem(idx, 4) * SC, SC)

        def rdma(src, dst, ssem, rsem, dev):
            return pltpu.make_async_remote_copy(
                src_ref=src, dst_ref=dst, send_sem=ssem, recv_sem=rsem,
                device_id=(dev,), device_id_type=pl.DeviceIdType.MESH,
            )

        barrier_sem = pltpu.get_barrier_semaphore()
        for nbr in (next_q, prev_q, next_r, prev_r):
            pl.semaphore_signal(
                barrier_sem, inc=1,
                device_id=(nbr,), device_id_type=pl.DeviceIdType.MESH,
            )
        pl.semaphore_wait(barrier_sem, 4)

        head0 = p * HQ_LOC
        copies = []
        for h in range(HQ_LOC):
            ck = pltpu.make_async_copy(
                k_hbm.at[0, :, head0 + h, :], k_buf.at[h], local_sems.at[h])
            cv = pltpu.make_async_copy(
                v_hbm.at[0, :, head0 + h, :], v_buf.at[h],
                local_sems.at[HQ_LOC + h])
            ck.start()
            cv.start()
            copies.append((ck, cv))

        q32 = jnp.dot(x_ref[0, 0:32, :], wq_ref[...],
                      preferred_element_type=jnp.float32)
        for ck, cv in copies:
            ck.wait()
            cv.wait()
        ctx32 = []
        for h in range(HQ_LOC):
            s32 = lax.dot_general(
                q32[:, h * DH:(h + 1) * DH], k_buf[h],
                dimension_numbers=(((1,), (1,)), ((), ())),
                preferred_element_type=jnp.float32) * SCALE
            s32 = s32 - jnp.max(s32, axis=-1, keepdims=True)
            e32 = jnp.exp(s32)
            e32 = e32 / jnp.sum(e32, axis=-1, keepdims=True)
            ctx32.append(jnp.dot(e32, v_buf[h],
                                 preferred_element_type=jnp.float32))
        pfix[...] = jnp.dot(jnp.concatenate(ctx32, axis=1), wo_ref[...],
                            preferred_element_type=jnp.float32)

        def compute_chunk(c):
            rows = pl.ds(c * PC, PC)
            q_chunk[...] = jnp.dot(
                x_ref[0, rows, :], wq_ref[...],
                preferred_element_type=jnp.float32)
            w0 = jnp.clip(c * PC - 128, 0, SKV - WIN)
            qi = lax.broadcasted_iota(jnp.int32, (PC, WIN), 0) + c * PC
            kw = lax.broadcasted_iota(jnp.int32, (PC, WIN), 1) + w0
            mask_w = (jnp.abs(qi - kw) <= 128) | (kw < 32)
            kg = lax.broadcasted_iota(jnp.int32, (PC, GB), 1)
            mask_g = (kg < 32) & (w0 > 0)
            for h in range(HQ_LOC):
                qh = q_chunk[:, h * DH:(h + 1) * DH]
                sw = lax.dot_general(
                    qh, k_buf[h, pl.ds(w0, WIN), :],
                    dimension_numbers=(((1,), (1,)), ((), ())),
                    preferred_element_type=jnp.float32) * SCALE
                sg = lax.dot_general(
                    qh, k_buf[h, 0:GB, :],
                    dimension_numbers=(((1,), (1,)), ((), ())),
                    preferred_element_type=jnp.float32) * SCALE
                sw = jnp.where(mask_w, sw, NEG)
                sg = jnp.where(mask_g, sg, NEG)
                m = jnp.maximum(jnp.max(sw, axis=-1, keepdims=True),
                                jnp.max(sg, axis=-1, keepdims=True))
                ew = jnp.exp(sw - m)
                eg = jnp.exp(sg - m)
                den = (jnp.sum(ew, axis=-1, keepdims=True)
                       + jnp.sum(eg, axis=-1, keepdims=True))
                ctx = (jnp.dot(ew, v_buf[h, pl.ds(w0, WIN), :],
                               preferred_element_type=jnp.float32)
                       + jnp.dot(eg, v_buf[h, 0:GB, :],
                                 preferred_element_type=jnp.float32)) / den
                ctx_chunk[:, h * DH:(h + 1) * DH] = ctx
            partial_buf[rows, :] = jnp.dot(
                ctx_chunk[...], wo_ref[...],
                preferred_element_type=jnp.float32)

            @pl.when(c == 0)
            def _():
                partial_buf[0:32, :] = pfix[...]

        compute_chunk(q)
        a_cw = [rdma(partial_buf.at[rows4(q), 0:CH], recv_a_cw.at[0],
                     a_send.at[CW, 0], a_recv.at[CW, 0], next_q)]
        a_cw[0].start()
        a_ccw = [rdma(partial_buf.at[rows4(q), CH:SQ], recv_a_ccw.at[0],
                      a_send.at[CCW, 0], a_recv.at[CCW, 0], prev_q)]
        a_ccw[0].start()

        compute_chunk(lax.rem(q + 3, 4))
        a_cw[0].wait()
        send_a_cw[1, :, :] = recv_a_cw[0] + partial_buf[rows4(q + 3), 0:CH]
        a_cw.append(rdma(send_a_cw.at[1], recv_a_cw.at[1],
                         a_send.at[CW, 1], a_recv.at[CW, 1], next_q))
        a_cw[1].start()

        compute_chunk(lax.rem(q + 1, 4))
        a_ccw[0].wait()
        send_a_ccw[1, :, :] = recv_a_ccw[0] + partial_buf[rows4(q + 1), CH:SQ]
        a_ccw.append(rdma(send_a_ccw.at[1], recv_a_ccw.at[1],
                          a_send.at[CCW, 1], a_recv.at[CCW, 1], prev_q))
        a_ccw[1].start()

        compute_chunk(lax.rem(q + 2, 4))
        a_cw[1].wait()
        send_a_cw[2, :, :] = recv_a_cw[1] + partial_buf[rows4(q + 2), 0:CH]
        a_cw.append(rdma(send_a_cw.at[2], recv_a_cw.at[2],
                         a_send.at[CW, 2], a_recv.at[CW, 2], next_q))
        a_cw[2].start()
        a_ccw[1].wait()
        send_a_ccw[2, :, :] = recv_a_ccw[1] + partial_buf[rows4(q + 2), CH:SQ]
        a_ccw.append(rdma(send_a_ccw.at[2], recv_a_ccw.at[2],
                          a_send.at[CCW, 2], a_recv.at[CCW, 2], prev_q))
        a_ccw[2].start()

        a_cw[2].wait()
        strip_cw[...] = recv_a_cw[2] + partial_buf[rows4(q + 1), 0:CH]
        a_ccw[2].wait()
        strip_ccw[...] = recv_a_ccw[2] + partial_buf[rows4(q + 3), CH:SQ]


        b_cw, b_ccw = [], []
        for t in range(3):
            if t == 0:
                src_cw = strip_cw.at[subrows(r), :]
                src_ccw = strip_ccw.at[subrows(r), :]
            else:
                b_cw[t - 1].wait()
                send_b_cw[t, :, :] = (
                    recv_b_cw[t - 1] + strip_cw[subrows(r + 4 - t), :])
                src_cw = send_b_cw.at[t]
                b_ccw[t - 1].wait()
                send_b_ccw[t, :, :] = (
                    recv_b_ccw[t - 1] + strip_ccw[subrows(r + t), :])
                src_ccw = send_b_ccw.at[t]
            rd = rdma(src_cw, recv_b_cw.at[t],
                      b_send.at[CW, t], b_recv.at[CW, t], next_r)
            rd.start()
            b_cw.append(rd)
            rd = rdma(src_ccw, recv_b_ccw.at[t],
                      b_send.at[CCW, t], b_recv.at[CCW, t], prev_r)
            rd.start()
            b_ccw.append(rd)
        b_cw[2].wait()
        strip_cw[subrows(r + 1), :] = (
            recv_b_cw[2] + strip_cw[subrows(r + 1), :])
        b_ccw[2].wait()
        strip_ccw[subrows(r + 3), :] = (
            recv_b_ccw[2] + strip_ccw[subrows(r + 3), :])

        for u in range(3):
            rd1 = rdma(strip_cw.at[subrows(r + 5 - u), :],
                       strip_cw.at[subrows(r + 5 - u), :],
                       c_send.at[CW, u], c_recv.at[CW, u], next_r)
            rd1.start()
            rd2 = rdma(strip_ccw.at[subrows(r + 3 + u), :],
                       strip_ccw.at[subrows(r + 3 + u), :],
                       c_send.at[CCW, u], c_recv.at[CCW, u], prev_r)
            rd2.start()
            rd1.wait()
            rd2.wait()

        out_ref[0, rows4(q + 1), 0:CH] = strip_cw[...]
        out_ref[0, rows4(q + 3), CH:SQ] = strip_ccw[...]

        for u in range(3):
            rd1 = rdma(out_ref.at[0, rows4(q + 5 - u), 0:CH],
                       out_ref.at[0, rows4(q + 5 - u), 0:CH],
                       d_send.at[CW, u], d_recv.at[CW, u], next_q)
            rd1.start()
            rd2 = rdma(out_ref.at[0, rows4(q + 3 + u), CH:SQ],
                       out_ref.at[0, rows4(q + 3 + u), CH:SQ],
                       d_send.at[CCW, u], d_recv.at[CCW, u], prev_q)
            rd2.start()
            rd1.wait()
            rd2.wait()

    return pl.pallas_call(
        body,
        out_shape=jax.ShapeDtypeStruct((1, SQ, SQ), jnp.float32),
        in_specs=[
            pl.BlockSpec(memory_space=pltpu.VMEM),
            pl.BlockSpec(memory_space=pltpu.VMEM),
            pl.BlockSpec(memory_space=pl.ANY),
            pl.BlockSpec(memory_space=pl.ANY),
            pl.BlockSpec(memory_space=pltpu.VMEM),
        ],
        out_specs=pl.BlockSpec(memory_space=pltpu.VMEM),
        scratch_shapes=[
            pltpu.VMEM((PC, D_LOC), jnp.float32),
            pltpu.VMEM((PC, D_LOC), jnp.float32),
            pltpu.VMEM((HQ_LOC, SKV, DH), jnp.float32),
            pltpu.VMEM((HQ_LOC, SKV, DH), jnp.float32),
            pltpu.VMEM((SQ, SQ), jnp.float32),
            pltpu.VMEM((32, SQ), jnp.float32),
            pltpu.VMEM((3, PC, CH), jnp.float32),
            pltpu.VMEM((3, PC, CH), jnp.float32),
            pltpu.VMEM((3, PC, CH), jnp.float32),
            pltpu.VMEM((3, PC, CH), jnp.float32),
            pltpu.VMEM((PC, CH), jnp.float32),
            pltpu.VMEM((PC, CH), jnp.float32),
            pltpu.VMEM((3, SC, CH), jnp.float32),
            pltpu.VMEM((3, SC, CH), jnp.float32),
            pltpu.VMEM((3, SC, CH), jnp.float32),
            pltpu.VMEM((3, SC, CH), jnp.float32),
            pltpu.SemaphoreType.DMA((2 * HQ_LOC,)),
            pltpu.SemaphoreType.DMA((2, 3)),
            pltpu.SemaphoreType.DMA((2, 3)),
            pltpu.SemaphoreType.DMA((2, 3)),
            pltpu.SemaphoreType.DMA((2, 3)),
            pltpu.SemaphoreType.DMA((2, 3)),
            pltpu.SemaphoreType.DMA((2, 3)),
            pltpu.SemaphoreType.DMA((2, 3)),
            pltpu.SemaphoreType.DMA((2, 3)),
        ],
        compiler_params=pltpu.CompilerParams(collective_id=0),
    )(x, Wq, K_ext, V_ext, Wo)
